# baseline (speedup 1.0000x reference)
import jax
import jax.numpy as jnp
from jax import lax
from jax.experimental import pallas as pl
from jax.experimental.pallas import tpu as pltpu
from jax.experimental.pallas import tpu_sc as plsc

ROWS, COLS = 4096, 200
EMB_DIM = 64
PAD_DIM = 128
VOCAB = 1000000
B = ROWS * COLS
NC, NS = 2, 16
NW = NC * NS
B_PER_W = B // NW
CHUNK = 128
N_CHUNKS = B_PER_W // CHUNK
NBUF = 8
N_OUTER = N_CHUNKS // NBUF


def _gather_body(x_hbm, t2_hbm, out_hbm, idx_v, rows_v, *sems):
    wid = lax.axis_index("s") * NC + lax.axis_index("c")
    base = wid * B_PER_W
    pltpu.sync_copy(x_hbm.at[pl.ds(wid * N_CHUNKS, N_CHUNKS)], idx_v)

    def fire(c, slot):
        return pltpu.async_copy(
            t2_hbm.at[idx_v.at[c]], rows_v.at[slot], sems[slot]
        )

    for b in range(NBUF - 1):
        fire(b, b)

    def outer(t, carry):
        for b in range(NBUF):
            c = t * NBUF + b
            nxt = c + NBUF - 1

            @pl.when(nxt < N_CHUNKS)
            def _():
                fire(nxt, (b + NBUF - 1) % NBUF)

            pltpu.make_async_copy(
                t2_hbm.at[idx_v.at[c]], rows_v.at[b], sems[b]
            ).wait()
            pltpu.sync_copy(
                rows_v.at[b],
                out_hbm.at[pl.ds(base + c * CHUNK, CHUNK), pl.ds(0, EMB_DIM)],
            )
        return carry

    lax.fori_loop(0, N_OUTER, outer, 0)


@jax.jit
def _embed(x2d, t2):
    mesh = plsc.VectorSubcoreMesh(core_axis_name="c", subcore_axis_name="s")
    gather = pl.kernel(
        _gather_body,
        out_type=jax.ShapeDtypeStruct((B, PAD_DIM), jnp.float32),
        mesh=mesh,
        scratch_types=[
            pltpu.VMEM((N_CHUNKS, CHUNK), jnp.int32),
            pltpu.VMEM((NBUF, CHUNK, EMB_DIM), jnp.float32),
        ]
        + [pltpu.SemaphoreType.DMA] * NBUF,
        compiler_params=pltpu.CompilerParams(use_tc_tiling_on_sc=False),
    )
    return gather(x2d, t2)


def kernel(x, table):
    x2d = x.reshape(B // CHUNK, CHUNK)
    n_split = 4
    h = VOCAB // n_split
    parts = lax.optimization_barrier(
        tuple(table[i * h:(i + 1) * h] for i in range(n_split))
    )
    t_u = jnp.concatenate(parts, axis=0)
    out = _embed(x2d, t_u)
    return out[:, :EMB_DIM].reshape(ROWS, COLS, EMB_DIM)


# speedup vs baseline: 1.7614x; 1.7614x over previous
import jax
import jax.numpy as jnp
from jax import lax
from jax.experimental import pallas as pl
from jax.experimental.pallas import tpu as pltpu
from jax.experimental.pallas import tpu_sc as plsc

ROWS, COLS = 4096, 200
EMB_DIM = 64
PAD_DIM = 128
VOCAB = 1000000
B = ROWS * COLS
NC, NS = 2, 16
NW = NC * NS
B_PER_W = B // NW
CHUNK = 128
N_CHUNKS = B_PER_W // CHUNK
NBUF = 8
N_OUTER = N_CHUNKS // NBUF


def _gather_body(x_hbm, t2_hbm, out_hbm, idx_v, rows_v, *sems):
    wid = lax.axis_index("s") * NC + lax.axis_index("c")
    base = wid * B_PER_W
    pltpu.sync_copy(x_hbm.at[pl.ds(wid * N_CHUNKS, N_CHUNKS)], idx_v)

    def fire(c, slot):
        return pltpu.async_copy(
            t2_hbm.at[idx_v.at[c]], rows_v.at[slot], sems[slot]
        )

    for b in range(NBUF - 1):
        fire(b, b)

    def outer(t, carry):
        for b in range(NBUF):
            c = t * NBUF + b
            nxt = c + NBUF - 1

            @pl.when(nxt < N_CHUNKS)
            def _():
                fire(nxt, (b + NBUF - 1) % NBUF)

            pltpu.make_async_copy(
                t2_hbm.at[idx_v.at[c]], rows_v.at[b], sems[b]
            ).wait()
            pltpu.sync_copy(
                rows_v.at[b],
                out_hbm.at[pl.ds(base + c * CHUNK, CHUNK), pl.ds(0, EMB_DIM)],
            )
        return carry

    lax.fori_loop(0, N_OUTER, outer, 0)


@jax.jit
def _embed(x2d, t2):
    mesh = plsc.VectorSubcoreMesh(core_axis_name="c", subcore_axis_name="s")
    gather = pl.kernel(
        _gather_body,
        out_type=jax.ShapeDtypeStruct((B, PAD_DIM), jnp.float32),
        mesh=mesh,
        scratch_types=[
            pltpu.VMEM((N_CHUNKS, CHUNK), jnp.int32),
            pltpu.VMEM((NBUF, CHUNK, EMB_DIM), jnp.float32),
        ]
        + [pltpu.SemaphoreType.DMA] * NBUF,
        compiler_params=pltpu.CompilerParams(use_tc_tiling_on_sc=False),
    )
    return gather(x2d, t2)


def kernel(x, table):
    x2d = x.reshape(B // CHUNK, CHUNK)
    out = _embed(x2d, table)
    return out[:, :EMB_DIM].reshape(ROWS, COLS, EMB_DIM)


# speedup vs baseline: 1.7657x; 1.0025x over previous
import jax
import jax.numpy as jnp
from jax import lax
from jax.experimental import pallas as pl
from jax.experimental.pallas import tpu as pltpu
from jax.experimental.pallas import tpu_sc as plsc

ROWS, COLS = 4096, 200
EMB_DIM = 64
PAD_DIM = 128
VOCAB = 1000000
B = ROWS * COLS
NC, NS = 2, 16
NW = NC * NS
B_PER_W = B // NW
CHUNK = 128
N_CHUNKS = B_PER_W // CHUNK
NBUF = 10
N_OUTER = N_CHUNKS // NBUF


def _gather_body(x_hbm, table_hbm, out_hbm, idx_v, rows_v, *sems):
    wid = lax.axis_index("s") * NC + lax.axis_index("c")
    base = wid * B_PER_W
    pltpu.sync_copy(x_hbm.at[pl.ds(wid * N_CHUNKS, N_CHUNKS)], idx_v)

    def fire(c, slot):
        return pltpu.async_copy(
            table_hbm.at[idx_v.at[c]], rows_v.at[slot], sems[slot]
        )

    for b in range(NBUF - 1):
        fire(b, b)

    def outer(t, carry):
        for b in range(NBUF):
            c = t * NBUF + b
            nxt = c + NBUF - 1

            @pl.when(nxt < N_CHUNKS)
            def _():
                fire(nxt, (b + NBUF - 1) % NBUF)

            pltpu.make_async_copy(
                table_hbm.at[idx_v.at[c]], rows_v.at[b], sems[b]
            ).wait()
            pltpu.sync_copy(
                rows_v.at[b],
                out_hbm.at[pl.ds(base + c * CHUNK, CHUNK), pl.ds(0, EMB_DIM)],
            )
        return carry

    lax.fori_loop(0, N_OUTER, outer, 0)


@jax.jit
def _embed(x2d, table):
    mesh = plsc.VectorSubcoreMesh(core_axis_name="c", subcore_axis_name="s")
    gather = pl.kernel(
        _gather_body,
        out_type=jax.ShapeDtypeStruct((B, PAD_DIM), jnp.float32),
        mesh=mesh,
        scratch_types=[
            pltpu.VMEM((N_CHUNKS, CHUNK), jnp.int32),
            pltpu.VMEM((NBUF, CHUNK, EMB_DIM), jnp.float32),
        ]
        + [pltpu.SemaphoreType.DMA] * NBUF,
        compiler_params=pltpu.CompilerParams(use_tc_tiling_on_sc=False),
    )
    return gather(x2d, table)


def kernel(x, table):
    x2d = x.reshape(B // CHUNK, CHUNK)
    out = _embed(x2d, table)
    return out[:, :EMB_DIM].reshape(ROWS, COLS, EMB_DIM)
